# speedup vs baseline: 1.2640x; 1.2640x over previous
import jax
import jax.numpy as jnp
from jax.experimental import pallas as pl
from jax.experimental.pallas import tpu as pltpu

B, R, C = 4, 8192, 1024
_ROW_BLK_NORM = 512
_ROW_BLK_MUL = 256


def _norms_body(w_ref, mag_ref):
    x = w_ref[...]
    mag_ref[...] = jnp.sqrt(jnp.sum(x * x, axis=-1))


def _select_body(mag_ref, logits_ref, wt_ref, b_ref, mask_ref):
    logit = jnp.sum(logits_ref[...] * wt_ref[...], axis=-1, keepdims=True)
    logit = logit + b_ref[0, 0]
    kf = jax.nn.sigmoid(logit)
    k = jnp.clip((kf * R).astype(jnp.int32), 1, None)

    bits = pltpu.bitcast(mag_ref[...], jnp.int32)

    def step(i, prefix):
        bit = 30 - i
        trial = prefix | (1 << bit)
        cnt = jnp.sum((bits >= trial).astype(jnp.int32), axis=-1,
                      keepdims=True)
        return jnp.where(cnt >= k, trial, prefix)

    prefix = jax.lax.fori_loop(0, 31, step, jnp.zeros((B, 1), jnp.int32))
    mask_ref[...] = (bits >= prefix).astype(jnp.float32)


def _mul_body(w_ref, mask_ref, out_ref):
    out_ref[...] = w_ref[...] * mask_ref[...][:, :, None]


@jax.jit
def kernel(weight_params, logits, W, b):
    wt = W.reshape(1, C)
    b2 = b.reshape(1, 1)

    mags = pl.pallas_call(
        _norms_body,
        grid=(R // _ROW_BLK_NORM,),
        in_specs=[pl.BlockSpec((B, _ROW_BLK_NORM, C), lambda r: (0, r, 0))],
        out_specs=pl.BlockSpec((B, _ROW_BLK_NORM), lambda r: (0, r)),
        out_shape=jax.ShapeDtypeStruct((B, R), jnp.float32),
    )(weight_params)

    mask = pl.pallas_call(
        _select_body,
        in_specs=[
            pl.BlockSpec((B, R), lambda: (0, 0)),
            pl.BlockSpec((B, C), lambda: (0, 0)),
            pl.BlockSpec((1, C), lambda: (0, 0)),
            pl.BlockSpec(memory_space=pltpu.SMEM),
        ],
        out_specs=pl.BlockSpec((B, R), lambda: (0, 0)),
        out_shape=jax.ShapeDtypeStruct((B, R), jnp.float32),
    )(mags, logits, wt, b2)

    out = pl.pallas_call(
        _mul_body,
        grid=(R // _ROW_BLK_MUL,),
        in_specs=[
            pl.BlockSpec((B, _ROW_BLK_MUL, C), lambda r: (0, r, 0)),
            pl.BlockSpec((B, _ROW_BLK_MUL), lambda r: (0, r)),
        ],
        out_specs=pl.BlockSpec((B, _ROW_BLK_MUL, C), lambda r: (0, r, 0)),
        out_shape=jax.ShapeDtypeStruct((B, R, C), jnp.float32),
    )(weight_params, mask)

    return out
